# Initial kernel scaffold; baseline (speedup 1.0000x reference)
import jax
import jax.numpy as jnp
from jax.experimental import pallas as pl


def kernel(x, edge_index, pseudo, weight, root_weight, bias):
    raise NotImplementedError("write your pallas kernel here")


# speedup vs baseline: 1.0587x; 1.0587x over previous
import functools

import jax
import jax.numpy as jnp
from jax import lax
from jax.experimental import pallas as pl
from jax.experimental.pallas import tpu as pltpu
from jax.experimental.pallas import tpu_sc as plsc

N = 10000
E = 320000
F = 128
FH = F // 2
KPROD = 25
KS = 5

NC, NS = 2, 16
EPT = E // NS
EST = EPT // 2
C = 80
NCHUNK = EST // C
N_PAD = 10240
RPT = N_PAD // NS


def _xt_body(x_ref, w_ref, o_ref):
    o_ref[0] = jnp.dot(x_ref[...], w_ref[0], preferred_element_type=jnp.float32)


def _compute_xt(x, weight):
    nb = 5
    bn = N // nb
    return pl.pallas_call(
        _xt_body,
        grid=(nb, KPROD),
        in_specs=[
            pl.BlockSpec((bn, F), lambda i, k: (i, 0)),
            pl.BlockSpec((1, F, F), lambda i, k: (k, 0, 0)),
        ],
        out_specs=pl.BlockSpec((1, bn, F), lambda i, k: (k, i, 0)),
        out_shape=jax.ShapeDtypeStruct((KPROD, N, F), jnp.float32),
    )(x, weight)


def _sc_edges(xt_half, row, col, p0, p1):
    mesh = plsc.VectorSubcoreMesh(core_axis_name="c", subcore_axis_name="s",
                                  num_cores=NC, num_subcores=NS)

    @functools.partial(
        pl.kernel,
        mesh=mesh,
        out_type=[
            jax.ShapeDtypeStruct((NC, N_PAD, FH), jnp.float32),
            jax.ShapeDtypeStruct((NC, N_PAD, 16), jnp.float32),
        ],
        scratch_types=[
            pltpu.VMEM((EST,), jnp.int32),
            pltpu.VMEM((EST,), jnp.int32),
            pltpu.VMEM((EST + 16,), jnp.float32),
            pltpu.VMEM((EST + 16,), jnp.float32),
            pltpu.VMEM((4, C), jnp.int32),
            pltpu.VMEM((1, C), jnp.int32),
            pltpu.VMEM((4, C, FH), jnp.float32),
            pltpu.VMEM((C, FH), jnp.float32),
            pltpu.VMEM((C, 16), jnp.float32),
            pltpu.VMEM((RPT, 16), jnp.float32),
            pltpu.VMEM_SHARED((N_PAD, FH), jnp.float32),
            pltpu.VMEM_SHARED((N_PAD, 16), jnp.float32),
            pltpu.SemaphoreType.DMA,
        ],
        compiler_params=pltpu.CompilerParams(use_tc_tiling_on_sc=False),
    )
    def k(xt_ref, row_ref, col_ref, p0_ref, p1_ref, out_ref, deg_ref,
          rowv, colv, p0v, p1v, gidx, ridx, rows, outb, onesb,
          dzb, acc, dacc, sem):
        cid = lax.axis_index("c")
        sid = lax.axis_index("s")
        base_e = sid * EPT

        z16 = jnp.zeros((16,), jnp.float32)
        o16 = jnp.ones((16,), jnp.float32)

        def zloop(r, _):
            for cc in range(FH // 16):
                outb[r, pl.ds(cc * 16, 16)] = z16
            return 0
        lax.fori_loop(0, C, zloop, 0)

        def dzloop(r, _):
            dzb[r, :] = z16
            return 0
        lax.fori_loop(0, RPT, dzloop, 0)

        def oloop(r, _):
            onesb[r, :] = o16
            return 0
        lax.fori_loop(0, C, oloop, 0)

        for j in range(RPT // C):
            pltpu.sync_copy(outb, acc.at[pl.ds(sid * RPT + j * C, C)])
        pltpu.sync_copy(dzb, dacc.at[pl.ds(sid * RPT, RPT)])
        plsc.subcore_barrier()

        def half_body(h, _):
            hb = base_e + h * EST
            pltpu.sync_copy(row_ref.at[pl.ds(hb, EST)], rowv)
            pltpu.sync_copy(col_ref.at[pl.ds(hb, EST)], colv)
            pltpu.sync_copy(p0_ref.at[pl.ds(hb, EST)], p0v.at[pl.ds(0, EST)])
            pltpu.sync_copy(p1_ref.at[pl.ds(hb, EST)], p1v.at[pl.ds(0, EST)])

            def chunk_body(g, _):
                e0 = g * C
                for j in range(C // 16):
                    off = e0 + j * 16
                    sl = pl.ds(j * 16, 16)
                    col16 = colv[pl.ds(off, 16)]
                    v0 = p0v[pl.ds(off, 16)] * 4.0
                    v1 = p1v[pl.ds(off, 16)] * 4.0
                    b0 = v0.astype(jnp.int32)
                    b1 = v1.astype(jnp.int32)
                    i00 = jnp.clip(b0, 0, KS - 1)
                    i01 = jnp.clip(b0 + 1, 0, KS - 1)
                    i10 = jnp.clip(b1, 0, KS - 1)
                    i11 = jnp.clip(b1 + 1, 0, KS - 1)
                    gidx[0, sl] = ((i00 + KS * i10) * N + col16) * 2 + cid
                    gidx[1, sl] = ((i01 + KS * i10) * N + col16) * 2 + cid
                    gidx[2, sl] = ((i00 + KS * i11) * N + col16) * 2 + cid
                    gidx[3, sl] = ((i01 + KS * i11) * N + col16) * 2 + cid
                    ridx[0, sl] = rowv[pl.ds(off, 16)]

                cps = [pltpu.async_copy(xt_ref.at[gidx.at[s]], rows.at[s], sem)
                       for s in range(4)]
                for cp in cps:
                    cp.wait()

                def edge_body(e, _):
                    v0 = jnp.full((16,), p0v[pl.ds(e0 + e, 16)][0]) * 4.0
                    t0 = v0 - v0.astype(jnp.int32).astype(jnp.float32)
                    v1 = jnp.full((16,), p1v[pl.ds(e0 + e, 16)][0]) * 4.0
                    t1 = v1 - v1.astype(jnp.int32).astype(jnp.float32)
                    for cc in range(FH // 16):
                        fl = pl.ds(cc * 16, 16)
                        r00 = rows[0, e, fl]
                        r01 = rows[1, e, fl]
                        r10 = rows[2, e, fl]
                        r11 = rows[3, e, fl]
                        a = r00 + t0 * (r01 - r00)
                        b = r10 + t0 * (r11 - r10)
                        outb[e, fl] = a + t1 * (b - a)
                    return 0
                lax.fori_loop(0, C, edge_body, 0)

                pltpu.sync_copy(outb, acc.at[ridx.at[0]], add=True)
                pltpu.sync_copy(onesb, dacc.at[ridx.at[0]], add=True)
                return 0

            lax.fori_loop(0, NCHUNK, chunk_body, 0)
            return 0

        lax.fori_loop(0, 2, half_body, 0)

        plsc.subcore_barrier()
        pltpu.sync_copy(acc.at[pl.ds(sid * RPT, RPT)],
                        out_ref.at[cid, pl.ds(sid * RPT, RPT)])
        pltpu.sync_copy(dacc.at[pl.ds(sid * RPT, RPT)],
                        deg_ref.at[cid, pl.ds(sid * RPT, RPT)])

    return k(xt_half, row, col, p0, p1)


def _final_body(p_ref, dg_ref, x_ref, rw_ref, b_ref, o_ref):
    psum = jnp.concatenate([p_ref[0], p_ref[1]], axis=-1)
    d = jnp.maximum(dg_ref[0, :, 0:1], 1.0)
    root = jnp.dot(x_ref[...], rw_ref[...], preferred_element_type=jnp.float32)
    o_ref[...] = psum / d + root + b_ref[...]


def _finalize(partials, deg, x, root_weight, bias2d):
    nb = 5
    bn = N // nb
    return pl.pallas_call(
        _final_body,
        grid=(nb,),
        in_specs=[
            pl.BlockSpec((NC, bn, FH), lambda i: (0, i, 0)),
            pl.BlockSpec((1, bn, 16), lambda i: (0, i, 0)),
            pl.BlockSpec((bn, F), lambda i: (i, 0)),
            pl.BlockSpec((F, F), lambda i: (0, 0)),
            pl.BlockSpec((1, F), lambda i: (0, 0)),
        ],
        out_specs=pl.BlockSpec((bn, F), lambda i: (i, 0)),
        out_shape=jax.ShapeDtypeStruct((N, F), jnp.float32),
    )(partials, deg, x, root_weight, bias2d)


def kernel(x, edge_index, pseudo, weight, root_weight, bias):
    xt = _compute_xt(x, weight)
    xt_half = xt.reshape(KPROD * N * 2, FH)
    row = edge_index[0]
    col = edge_index[1]
    pt = pseudo.T
    p0 = pt[0]
    p1 = pt[1]
    partials, deg = _sc_edges(xt_half, row, col, p0, p1)
    return _finalize(partials, deg, x, root_weight, bias.reshape(1, F))


# speedup vs baseline: 1.3950x; 1.3176x over previous
import functools

import jax
import jax.numpy as jnp
from jax import lax
from jax.experimental import pallas as pl
from jax.experimental.pallas import tpu as pltpu
from jax.experimental.pallas import tpu_sc as plsc

N = 10000
E = 320000
F = 128
FH = F // 2
KPROD = 25
KS = 5

NC, NS = 2, 16
EPT = E // NS
EST = 2000
C = 80
NCHUNK = EST // C
N_PAD = 10240
RPT = N_PAD // NS


def _xt_body(x_ref, w_ref, o_ref):
    o_ref[0] = jnp.dot(x_ref[...], w_ref[0], preferred_element_type=jnp.float32)


def _compute_xt(x, weight):
    nb = 5
    bn = N // nb
    return pl.pallas_call(
        _xt_body,
        grid=(nb, KPROD),
        in_specs=[
            pl.BlockSpec((bn, F), lambda i, k: (i, 0)),
            pl.BlockSpec((1, F, F), lambda i, k: (k, 0, 0)),
        ],
        out_specs=pl.BlockSpec((1, bn, F), lambda i, k: (k, i, 0)),
        out_shape=jax.ShapeDtypeStruct((KPROD, N, F), jnp.float32),
    )(x, weight)


def _sc_edges(xt_half, row, col, p0, p1):
    mesh = plsc.VectorSubcoreMesh(core_axis_name="c", subcore_axis_name="s",
                                  num_cores=NC, num_subcores=NS)

    @functools.partial(
        pl.kernel,
        mesh=mesh,
        out_type=[
            jax.ShapeDtypeStruct((NC, N_PAD, FH), jnp.float32),
            jax.ShapeDtypeStruct((NC, N_PAD, 16), jnp.float32),
        ],
        scratch_types=[
            pltpu.VMEM((EST,), jnp.int32),
            pltpu.VMEM((EST,), jnp.int32),
            pltpu.VMEM((EST,), jnp.float32),
            pltpu.VMEM((EST,), jnp.float32),
            pltpu.VMEM((4, C), jnp.int32),
            pltpu.VMEM((4, C), jnp.int32),
            pltpu.VMEM((1, C), jnp.int32),
            pltpu.VMEM((1, C), jnp.int32),
            pltpu.VMEM((C + 16,), jnp.float32),
            pltpu.VMEM((C + 16,), jnp.float32),
            pltpu.VMEM((C + 16,), jnp.float32),
            pltpu.VMEM((C + 16,), jnp.float32),
            pltpu.VMEM((4, C, FH), jnp.float32),
            pltpu.VMEM((4, C, FH), jnp.float32),
            pltpu.VMEM((C, FH), jnp.float32),
            pltpu.VMEM((C, 16), jnp.float32),
            pltpu.VMEM_SHARED((N_PAD, FH), jnp.float32),
            pltpu.VMEM_SHARED((N_PAD, 16), jnp.float32),
            pltpu.SemaphoreType.DMA,
            pltpu.SemaphoreType.DMA,
        ],
        compiler_params=pltpu.CompilerParams(use_tc_tiling_on_sc=False),
    )
    def k(xt_ref, row_ref, col_ref, p0_ref, p1_ref, out_ref, deg_ref,
          rowv, colv, p0v, p1v, gidxA, gidxB, ridxA, ridxB,
          f0A, f1A, f0B, f1B, rowsA, rowsB, outb, onesb,
          acc, dacc, gsemA, gsemB):
        cid = lax.axis_index("c")
        sid = lax.axis_index("s")
        base_e = sid * EPT

        z16 = jnp.zeros((16,), jnp.float32)
        o16 = jnp.ones((16,), jnp.float32)

        def zloop(r, _):
            for cc in range(FH // 16):
                outb[r, pl.ds(cc * 16, 16)] = z16
            return 0
        lax.fori_loop(0, C, zloop, 0)

        def ozloop(r, _):
            onesb[r, :] = z16
            return 0
        lax.fori_loop(0, C, ozloop, 0)

        for j in range(RPT // C):
            pltpu.sync_copy(outb, acc.at[pl.ds(sid * RPT + j * C, C)])
            pltpu.sync_copy(onesb, dacc.at[pl.ds(sid * RPT + j * C, C)])

        def oloop(r, _):
            onesb[r, :] = o16
            return 0
        lax.fori_loop(0, C, oloop, 0)
        plsc.subcore_barrier()

        def build(q, gidxX, ridxX, f0X, f1X):
            e0 = q * C
            for j in range(C // 16):
                off = e0 + j * 16
                sl = pl.ds(j * 16, 16)
                col16 = colv[pl.ds(off, 16)]
                v0 = p0v[pl.ds(off, 16)] * 4.0
                v1 = p1v[pl.ds(off, 16)] * 4.0
                b0 = v0.astype(jnp.int32)
                b1 = v1.astype(jnp.int32)
                f0X[sl] = v0 - b0.astype(jnp.float32)
                f1X[sl] = v1 - b1.astype(jnp.float32)
                i00 = jnp.clip(b0, 0, KS - 1)
                i01 = jnp.clip(b0 + 1, 0, KS - 1)
                i10 = jnp.clip(b1, 0, KS - 1)
                i11 = jnp.clip(b1 + 1, 0, KS - 1)
                gidxX[0, sl] = ((i00 + KS * i10) * N + col16) * 2 + cid
                gidxX[1, sl] = ((i01 + KS * i10) * N + col16) * 2 + cid
                gidxX[2, sl] = ((i00 + KS * i11) * N + col16) * 2 + cid
                gidxX[3, sl] = ((i01 + KS * i11) * N + col16) * 2 + cid
                ridxX[0, sl] = rowv[pl.ds(off, 16)]

        def fire(gidxX, rowsX, gsemX):
            return [pltpu.async_copy(xt_ref.at[gidxX.at[s]], rowsX.at[s], gsemX)
                    for s in range(4)]

        def wait_recon(gidxX, rowsX, gsemX):
            for s in range(4):
                pltpu.make_async_copy(xt_ref.at[gidxX.at[s]], rowsX.at[s],
                                      gsemX).wait()

        def compute(rowsX, f0X, f1X):
            def edge_body(e, _):
                t0 = jnp.full((16,), f0X[pl.ds(e, 16)][0])
                t1 = jnp.full((16,), f1X[pl.ds(e, 16)][0])
                for cc in range(FH // 16):
                    fl = pl.ds(cc * 16, 16)
                    r00 = rowsX[0, e, fl]
                    r01 = rowsX[1, e, fl]
                    r10 = rowsX[2, e, fl]
                    r11 = rowsX[3, e, fl]
                    a = r00 + t0 * (r01 - r00)
                    b = r10 + t0 * (r11 - r10)
                    outb[e, fl] = a + t1 * (b - a)
                return 0
            lax.fori_loop(0, C, edge_body, 0)

        def scatter(ridxX):
            pltpu.sync_copy(outb, acc.at[ridxX.at[0]], add=True)
            pltpu.sync_copy(onesb, dacc.at[ridxX.at[0]], add=True)

        def half_body(h, _):
            hb = base_e + h * EST
            pltpu.sync_copy(row_ref.at[pl.ds(hb, EST)], rowv)
            pltpu.sync_copy(col_ref.at[pl.ds(hb, EST)], colv)
            pltpu.sync_copy(p0_ref.at[pl.ds(hb, EST)], p0v)
            pltpu.sync_copy(p1_ref.at[pl.ds(hb, EST)], p1v)

            build(0, gidxA, ridxA, f0A, f1A)
            fire(gidxA, rowsA, gsemA)

            def pair_body(i, _):
                q = 2 * i
                wait_recon(gidxA, rowsA, gsemA)
                build(q + 1, gidxB, ridxB, f0B, f1B)
                fire(gidxB, rowsB, gsemB)
                compute(rowsA, f0A, f1A)
                scatter(ridxA)

                wait_recon(gidxB, rowsB, gsemB)
                build(q + 2, gidxA, ridxA, f0A, f1A)
                fire(gidxA, rowsA, gsemA)
                compute(rowsB, f0B, f1B)
                scatter(ridxB)
                return 0

            lax.fori_loop(0, (NCHUNK - 1) // 2, pair_body, 0)

            wait_recon(gidxA, rowsA, gsemA)
            compute(rowsA, f0A, f1A)
            scatter(ridxA)
            return 0

        lax.fori_loop(0, EPT // EST, half_body, 0)

        plsc.subcore_barrier()
        pltpu.sync_copy(acc.at[pl.ds(sid * RPT, RPT)],
                        out_ref.at[cid, pl.ds(sid * RPT, RPT)])
        pltpu.sync_copy(dacc.at[pl.ds(sid * RPT, RPT)],
                        deg_ref.at[cid, pl.ds(sid * RPT, RPT)])

    return k(xt_half, row, col, p0, p1)


def _final_body(p_ref, dg_ref, x_ref, rw_ref, b_ref, o_ref):
    psum = jnp.concatenate([p_ref[0], p_ref[1]], axis=-1)
    d = jnp.maximum(dg_ref[0, :, 0:1], 1.0)
    root = jnp.dot(x_ref[...], rw_ref[...], preferred_element_type=jnp.float32)
    o_ref[...] = psum / d + root + b_ref[...]


def _finalize(partials, deg, x, root_weight, bias2d):
    nb = 5
    bn = N // nb
    return pl.pallas_call(
        _final_body,
        grid=(nb,),
        in_specs=[
            pl.BlockSpec((NC, bn, FH), lambda i: (0, i, 0)),
            pl.BlockSpec((1, bn, 16), lambda i: (0, i, 0)),
            pl.BlockSpec((bn, F), lambda i: (i, 0)),
            pl.BlockSpec((F, F), lambda i: (0, 0)),
            pl.BlockSpec((1, F), lambda i: (0, 0)),
        ],
        out_specs=pl.BlockSpec((bn, F), lambda i: (i, 0)),
        out_shape=jax.ShapeDtypeStruct((N, F), jnp.float32),
    )(partials, deg, x, root_weight, bias2d)


def kernel(x, edge_index, pseudo, weight, root_weight, bias):
    xt = _compute_xt(x, weight)
    xt_half = xt.reshape(KPROD * N * 2, FH)
    row = edge_index[0]
    col = edge_index[1]
    pt = pseudo.T
    p0 = pt[0]
    p1 = pt[1]
    partials, deg = _sc_edges(xt_half, row, col, p0, p1)
    return _finalize(partials, deg, x, root_weight, bias.reshape(1, F))


# speedup vs baseline: 1.4945x; 1.0713x over previous
import functools

import jax
import jax.numpy as jnp
from jax import lax
from jax.experimental import pallas as pl
from jax.experimental.pallas import tpu as pltpu
from jax.experimental.pallas import tpu_sc as plsc

N = 10000
E = 320000
F = 128
FH = F // 2
KPROD = 25
KS = 5

NC, NS = 2, 16
EPT = E // NS
EST = 2000
C = 80
NCHUNK = EST // C
N_PAD = 10240
RPT = N_PAD // NS


def _xt_body(x_ref, w_ref, o_ref):
    o_ref[0] = jnp.dot(x_ref[...], w_ref[0], preferred_element_type=jnp.float32)


def _compute_xt(x, weight):
    nb = 5
    bn = N // nb
    return pl.pallas_call(
        _xt_body,
        grid=(nb, KPROD),
        in_specs=[
            pl.BlockSpec((bn, F), lambda i, k: (i, 0)),
            pl.BlockSpec((1, F, F), lambda i, k: (k, 0, 0)),
        ],
        out_specs=pl.BlockSpec((1, bn, F), lambda i, k: (k, i, 0)),
        out_shape=jax.ShapeDtypeStruct((KPROD, N, F), jnp.float32),
    )(x, weight)


def _sc_edges(xt_half, row, col, p0, p1):
    mesh = plsc.VectorSubcoreMesh(core_axis_name="c", subcore_axis_name="s",
                                  num_cores=NC, num_subcores=NS)

    @functools.partial(
        pl.kernel,
        mesh=mesh,
        out_type=[
            jax.ShapeDtypeStruct((NC, N_PAD, FH), jnp.float32),
            jax.ShapeDtypeStruct((NC, N_PAD, 16), jnp.float32),
        ],
        scratch_types=[
            pltpu.VMEM((EST,), jnp.int32),
            pltpu.VMEM((EST,), jnp.int32),
            pltpu.VMEM((EST,), jnp.float32),
            pltpu.VMEM((EST,), jnp.float32),
            pltpu.VMEM((1, 4 * C), jnp.int32),
            pltpu.VMEM((1, 4 * C), jnp.int32),
            pltpu.VMEM((4, C), jnp.int32),
            pltpu.VMEM((C + 16,), jnp.float32),
            pltpu.VMEM((C + 16,), jnp.float32),
            pltpu.VMEM((C + 16,), jnp.float32),
            pltpu.VMEM((C + 16,), jnp.float32),
            pltpu.VMEM((4 * C, FH), jnp.float32),
            pltpu.VMEM((4 * C, FH), jnp.float32),
            pltpu.VMEM((C, FH), jnp.float32),
            pltpu.VMEM((C, FH), jnp.float32),
            pltpu.VMEM((C, 16), jnp.float32),
            pltpu.VMEM_SHARED((N_PAD, FH), jnp.float32),
            pltpu.VMEM_SHARED((N_PAD, 16), jnp.float32),
            pltpu.SemaphoreType.DMA,
            pltpu.SemaphoreType.DMA,
            pltpu.SemaphoreType.DMA,
            pltpu.SemaphoreType.DMA,
        ],
        compiler_params=pltpu.CompilerParams(use_tc_tiling_on_sc=False),
    )
    def k(xt_ref, row_ref, col_ref, p0_ref, p1_ref, out_ref, deg_ref,
          rowv, colv, p0v, p1v, gidxA, gidxB, ridx4,
          f0A, f1A, f0B, f1B, rowsA, rowsB, outbA, outbB, onesb,
          acc, dacc, gsemA, gsemB, ssemA, ssemB):
        cid = lax.axis_index("c")
        sid = lax.axis_index("s")
        base_e = sid * EPT

        z16 = jnp.zeros((16,), jnp.float32)
        o16 = jnp.ones((16,), jnp.float32)

        def zloop(r, _):
            for cc in range(FH // 16):
                outbA[r, pl.ds(cc * 16, 16)] = z16
            return 0
        lax.fori_loop(0, C, zloop, 0)

        def ozloop(r, _):
            onesb[r, :] = z16
            return 0
        lax.fori_loop(0, C, ozloop, 0)

        for j in range(RPT // C):
            pltpu.sync_copy(outbA, acc.at[pl.ds(sid * RPT + j * C, C)])
            pltpu.sync_copy(onesb, dacc.at[pl.ds(sid * RPT + j * C, C)])

        def oloop(r, _):
            onesb[r, :] = o16
            return 0
        lax.fori_loop(0, C, oloop, 0)
        plsc.subcore_barrier()

        def build(q, gidxX, f0X, f1X):
            e0 = q * C
            for j in range(C // 16):
                off = e0 + j * 16
                sl = pl.ds(j * 16, 16)
                col16 = colv[pl.ds(off, 16)]
                v0 = p0v[pl.ds(off, 16)] * 4.0
                v1 = p1v[pl.ds(off, 16)] * 4.0
                b0 = v0.astype(jnp.int32)
                b1 = v1.astype(jnp.int32)
                f0X[sl] = v0 - b0.astype(jnp.float32)
                f1X[sl] = v1 - b1.astype(jnp.float32)
                i00 = jnp.clip(b0, 0, KS - 1)
                i01 = jnp.clip(b0 + 1, 0, KS - 1)
                i10 = jnp.clip(b1, 0, KS - 1)
                i11 = jnp.clip(b1 + 1, 0, KS - 1)
                gidxX[0, pl.ds(0 * C + j * 16, 16)] = ((i00 + KS * i10) * N + col16) * 2 + cid
                gidxX[0, pl.ds(1 * C + j * 16, 16)] = ((i01 + KS * i10) * N + col16) * 2 + cid
                gidxX[0, pl.ds(2 * C + j * 16, 16)] = ((i00 + KS * i11) * N + col16) * 2 + cid
                gidxX[0, pl.ds(3 * C + j * 16, 16)] = ((i01 + KS * i11) * N + col16) * 2 + cid
                ridx4[q & 3, sl] = rowv[pl.ds(off, 16)]

        def fire_gather(gidxX, rowsX, gsemX):
            pltpu.async_copy(xt_ref.at[gidxX.at[0]], rowsX, gsemX)

        def wait_gather(gidxX, rowsX, gsemX):
            pltpu.make_async_copy(xt_ref.at[gidxX.at[0]], rowsX, gsemX).wait()

        def compute(rowsX, f0X, f1X, outbX):
            def edge_body(e, _):
                t0 = jnp.full((16,), f0X[pl.ds(e, 16)][0])
                t1 = jnp.full((16,), f1X[pl.ds(e, 16)][0])
                for cc in range(FH // 16):
                    fl = pl.ds(cc * 16, 16)
                    r00 = rowsX[0 * C + e, fl]
                    r01 = rowsX[1 * C + e, fl]
                    r10 = rowsX[2 * C + e, fl]
                    r11 = rowsX[3 * C + e, fl]
                    a = r00 + t0 * (r01 - r00)
                    b = r10 + t0 * (r11 - r10)
                    outbX[e, fl] = a + t1 * (b - a)
                return 0
            lax.fori_loop(0, C, edge_body, 0)

        def fire_scatter(q, outbX, ssemX):
            pltpu.async_copy(outbX, acc.at[ridx4.at[q & 3]], ssemX, add=True)
            pltpu.async_copy(onesb, dacc.at[ridx4.at[q & 3]], ssemX, add=True)

        def wait_scatter(outbX, ssemX):
            pltpu.make_async_copy(outbX, acc.at[ridx4.at[0]], ssemX).wait()
            pltpu.make_async_copy(onesb, dacc.at[ridx4.at[0]], ssemX).wait()

        def half_body(h, _):
            hb = base_e + h * EST
            pltpu.sync_copy(row_ref.at[pl.ds(hb, EST)], rowv)
            pltpu.sync_copy(col_ref.at[pl.ds(hb, EST)], colv)
            pltpu.sync_copy(p0_ref.at[pl.ds(hb, EST)], p0v)
            pltpu.sync_copy(p1_ref.at[pl.ds(hb, EST)], p1v)

            build(0, gidxA, f0A, f1A)
            fire_gather(gidxA, rowsA, gsemA)

            def pair_body(i, _):
                q = 2 * i
                wait_gather(gidxA, rowsA, gsemA)
                build(q + 1, gidxB, f0B, f1B)
                fire_gather(gidxB, rowsB, gsemB)

                @pl.when(i >= 1)
                def _():
                    wait_scatter(outbA, ssemA)
                compute(rowsA, f0A, f1A, outbA)
                fire_scatter(q, outbA, ssemA)

                wait_gather(gidxB, rowsB, gsemB)
                build(q + 2, gidxA, f0A, f1A)
                fire_gather(gidxA, rowsA, gsemA)

                @pl.when(i >= 1)
                def _():
                    wait_scatter(outbB, ssemB)
                compute(rowsB, f0B, f1B, outbB)
                fire_scatter(q + 1, outbB, ssemB)
                return 0

            lax.fori_loop(0, (NCHUNK - 1) // 2, pair_body, 0)

            wait_gather(gidxA, rowsA, gsemA)
            wait_scatter(outbA, ssemA)
            compute(rowsA, f0A, f1A, outbA)
            fire_scatter(NCHUNK - 1, outbA, ssemA)
            wait_scatter(outbB, ssemB)
            wait_scatter(outbA, ssemA)
            return 0

        lax.fori_loop(0, EPT // EST, half_body, 0)

        plsc.subcore_barrier()
        pltpu.sync_copy(acc.at[pl.ds(sid * RPT, RPT)],
                        out_ref.at[cid, pl.ds(sid * RPT, RPT)])
        pltpu.sync_copy(dacc.at[pl.ds(sid * RPT, RPT)],
                        deg_ref.at[cid, pl.ds(sid * RPT, RPT)])

    return k(xt_half, row, col, p0, p1)


def _final_body(p_ref, dg_ref, x_ref, rw_ref, b_ref, o_ref):
    psum = jnp.concatenate([p_ref[0], p_ref[1]], axis=-1)
    d = jnp.maximum(dg_ref[0, :, 0:1], 1.0)
    root = jnp.dot(x_ref[...], rw_ref[...], preferred_element_type=jnp.float32)
    o_ref[...] = psum / d + root + b_ref[...]


def _finalize(partials, deg, x, root_weight, bias2d):
    nb = 5
    bn = N // nb
    return pl.pallas_call(
        _final_body,
        grid=(nb,),
        in_specs=[
            pl.BlockSpec((NC, bn, FH), lambda i: (0, i, 0)),
            pl.BlockSpec((1, bn, 16), lambda i: (0, i, 0)),
            pl.BlockSpec((bn, F), lambda i: (i, 0)),
            pl.BlockSpec((F, F), lambda i: (0, 0)),
            pl.BlockSpec((1, F), lambda i: (0, 0)),
        ],
        out_specs=pl.BlockSpec((bn, F), lambda i: (i, 0)),
        out_shape=jax.ShapeDtypeStruct((N, F), jnp.float32),
    )(partials, deg, x, root_weight, bias2d)


def kernel(x, edge_index, pseudo, weight, root_weight, bias):
    xt = _compute_xt(x, weight)
    xt_half = xt.reshape(KPROD * N * 2, FH)
    row = edge_index[0]
    col = edge_index[1]
    pt = pseudo.T
    p0 = pt[0]
    p1 = pt[1]
    partials, deg = _sc_edges(xt_half, row, col, p0, p1)
    return _finalize(partials, deg, x, root_weight, bias.reshape(1, F))
